# baseline (speedup 1.0000x reference)
import functools

import jax
import jax.numpy as jnp
from jax import lax
from jax.experimental import pallas as pl
from jax.experimental.pallas import tpu as pltpu
from jax.experimental.pallas import tpu_sc as plsc

N_NODES = 10000
D_OUT = 128
N_EDGES = 320000

NC = 2
NS = 16
NW = NC * NS
K = 128

CPW_A = 64
CPW_B = 96
NCHUNKS = NS * (CPW_A + CPW_B)
E_PAD = NCHUNKS * K

ACC_ROWS = 10240
SHARD = ACC_ROWS // NS
LAST_ROWS = N_NODES - (NS - 1) * SHARD


def _matmul_body(x_ref, w_ref, o_ref):
    o_ref[...] = jnp.dot(x_ref[...], w_ref[...],
                         preferred_element_type=jnp.float32)


def _matmul(X, W):
    n, d_in = X.shape
    d_out = W.shape[1]
    blk = 2000
    grid = n // blk
    return pl.pallas_call(
        _matmul_body,
        grid=(grid,),
        in_specs=[
            pl.BlockSpec((blk, d_in), lambda i: (i, 0)),
            pl.BlockSpec((d_in, d_out), lambda i: (0, 0)),
        ],
        out_specs=pl.BlockSpec((blk, d_out), lambda i: (i, 0)),
        out_shape=jax.ShapeDtypeStruct((n, d_out), jnp.float32),
    )(X, W)


def _sc_body(wh_hbm, edg_hbm, out_hbm, acc, idx_all, rows, gsem):
    cid = lax.axis_index("c")
    sid = lax.axis_index("s")

    wbase = jnp.where(cid == 0, sid * CPW_A, NS * CPW_A + sid * CPW_B)

    @pl.when(cid == 0)
    def _():
        pltpu.sync_copy(edg_hbm.at[pl.ds(wbase, CPW_A)],
                        idx_all.at[pl.ds(0, CPW_A)])

    @pl.when(cid == 1)
    def _():
        pltpu.sync_copy(edg_hbm.at[pl.ds(wbase, CPW_B)], idx_all)

    def _zero_row(i, carry):
        for c in range(D_OUT // 16):
            rows[i, pl.ds(c * 16, 16)] = jnp.zeros((16,), jnp.float32)
        return carry

    lax.fori_loop(0, K, _zero_row, 0)
    zbase = sid * SHARD
    for j in range(SHARD // K):
        pltpu.sync_copy(rows, acc.at[pl.ds(zbase + j * K, K), :])
    plsc.subcore_barrier()

    def _chunk(c, carry):
        pltpu.async_copy(wh_hbm.at[idx_all.at[c, 1]], rows, gsem).wait()
        pltpu.sync_copy(rows, acc.at[idx_all.at[c, 0]], add=True)
        return carry

    @pl.when(cid == 0)
    def _():
        lax.fori_loop(0, CPW_A, _chunk, 0)

    @pl.when(cid == 1)
    def _():
        lax.fori_loop(0, CPW_B, _chunk, 0)

    plsc.subcore_barrier()

    rb = sid * SHARD

    @pl.when(sid < NS - 1)
    def _():
        pltpu.sync_copy(acc.at[pl.ds(rb, SHARD), :],
                        out_hbm.at[cid, pl.ds(rb, SHARD), :])

    @pl.when(sid == NS - 1)
    def _():
        pltpu.sync_copy(acc.at[pl.ds(rb, LAST_ROWS), :],
                        out_hbm.at[cid, pl.ds(rb, LAST_ROWS), :])


_sc_scatter = functools.partial(
    pl.kernel,
    out_type=jax.ShapeDtypeStruct((NC, N_NODES, D_OUT), jnp.float32),
    mesh=plsc.VectorSubcoreMesh(core_axis_name="c", subcore_axis_name="s"),
    scratch_types=[
        pltpu.VMEM_SHARED((ACC_ROWS, D_OUT), jnp.float32),
        pltpu.VMEM((CPW_B, 2, K), jnp.int32),
        pltpu.VMEM((K, D_OUT), jnp.float32),
        pltpu.SemaphoreType.DMA,
    ],
)(_sc_body)


def _sum_body(p_ref, o_ref):
    o_ref[...] = p_ref[0] + p_ref[1]


def _sum2(parts):
    _, n, d = parts.shape
    blk = 2000
    return pl.pallas_call(
        _sum_body,
        grid=(n // blk,),
        in_specs=[pl.BlockSpec((NC, blk, d), lambda i: (0, i, 0))],
        out_specs=pl.BlockSpec((blk, d), lambda i: (i, 0)),
        out_shape=jax.ShapeDtypeStruct((n, d), jnp.float32),
    )(parts)


def kernel(X, edges, W, a):
    del a
    n = X.shape[0]
    e = edges.shape[1]
    Wh = _matmul(X, W)
    src = edges[0].astype(jnp.int32)
    tgt = edges[1].astype(jnp.int32)
    pad = E_PAD - e
    src = jnp.concatenate([src, jnp.full((pad,), n, jnp.int32)])
    tgt = jnp.concatenate([tgt, jnp.zeros((pad,), jnp.int32)])
    edg = jnp.stack([src.reshape(NCHUNKS, K), tgt.reshape(NCHUNKS, K)],
                    axis=1)
    parts = _sc_scatter(Wh, edg)
    return _sum2(parts)


# speedup vs baseline: 1.6390x; 1.6390x over previous
import functools

import jax
import jax.numpy as jnp
from jax import lax
from jax.experimental import pallas as pl
from jax.experimental.pallas import tpu as pltpu
from jax.experimental.pallas import tpu_sc as plsc

N_NODES = 10000
D_OUT = 128
N_EDGES = 320000

NC = 2
NS = 16
NW = NC * NS
K = 128

CPW = -(-N_EDGES // (NW * K))
NCHUNKS = CPW * NW
E_PAD = NCHUNKS * K

ACC_ROWS = 10240
SHARD = ACC_ROWS // NS
LAST_ROWS = N_NODES - (NS - 1) * SHARD


def _matmul_body(x_ref, w_ref, o_ref):
    o_ref[...] = jnp.dot(x_ref[...], w_ref[...],
                         preferred_element_type=jnp.float32)


def _matmul(X, W):
    n, d_in = X.shape
    d_out = W.shape[1]
    blk = 2000
    grid = n // blk
    return pl.pallas_call(
        _matmul_body,
        grid=(grid,),
        in_specs=[
            pl.BlockSpec((blk, d_in), lambda i: (i, 0)),
            pl.BlockSpec((d_in, d_out), lambda i: (0, 0)),
        ],
        out_specs=pl.BlockSpec((blk, d_out), lambda i: (i, 0)),
        out_shape=jax.ShapeDtypeStruct((n, d_out), jnp.float32),
    )(X, W)


def _sc_body(wh_hbm, edg_hbm, out_hbm, acc, idx_all, rows, gsem):
    cid = lax.axis_index("c")
    sid = lax.axis_index("s")
    wid = sid * NC + cid

    idx_cp = pltpu.async_copy(edg_hbm.at[wid], idx_all, gsem)

    def _zero_row(i, carry):
        for c in range(D_OUT // 16):
            rows[i, pl.ds(c * 16, 16)] = jnp.zeros((16,), jnp.float32)
        return carry

    lax.fori_loop(0, K, _zero_row, 0)
    zbase = sid * SHARD
    for j in range(SHARD // K):
        pltpu.sync_copy(rows, acc.at[pl.ds(zbase + j * K, K), :])
    plsc.subcore_barrier()

    idx_cp.wait()

    def _chunk(c, carry):
        pltpu.async_copy(wh_hbm.at[idx_all.at[c, 1]], rows, gsem).wait()
        pltpu.sync_copy(rows, acc.at[idx_all.at[c, 0]], add=True)
        return carry

    lax.fori_loop(0, CPW, _chunk, 0)
    plsc.subcore_barrier()

    rb = sid * SHARD

    @pl.when(sid < NS - 1)
    def _():
        pltpu.sync_copy(acc.at[pl.ds(rb, SHARD), :],
                        out_hbm.at[cid, pl.ds(rb, SHARD), :])

    @pl.when(sid == NS - 1)
    def _():
        pltpu.sync_copy(acc.at[pl.ds(rb, LAST_ROWS), :],
                        out_hbm.at[cid, pl.ds(rb, LAST_ROWS), :])


_sc_scatter = functools.partial(
    pl.kernel,
    out_type=jax.ShapeDtypeStruct((NC, N_NODES, D_OUT), jnp.float32),
    mesh=plsc.VectorSubcoreMesh(core_axis_name="c", subcore_axis_name="s"),
    scratch_types=[
        pltpu.VMEM_SHARED((ACC_ROWS, D_OUT), jnp.float32),
        pltpu.VMEM((CPW, 2, K), jnp.int32),
        pltpu.VMEM((K, D_OUT), jnp.float32),
        pltpu.SemaphoreType.DMA,
    ],
)(_sc_body)


def _sum_body(p_ref, o_ref):
    o_ref[...] = p_ref[0] + p_ref[1]


def _sum2(parts):
    _, n, d = parts.shape
    blk = 2000
    return pl.pallas_call(
        _sum_body,
        grid=(n // blk,),
        in_specs=[pl.BlockSpec((NC, blk, d), lambda i: (0, i, 0))],
        out_specs=pl.BlockSpec((blk, d), lambda i: (i, 0)),
        out_shape=jax.ShapeDtypeStruct((n, d), jnp.float32),
    )(parts)


def kernel(X, edges, W, a):
    del a
    n = X.shape[0]
    e = edges.shape[1]
    Wh = _matmul(X, W)
    src = edges[0].astype(jnp.int32)
    tgt = edges[1].astype(jnp.int32)
    pad = E_PAD - e
    src = jnp.concatenate([src, jnp.full((pad,), n, jnp.int32)])
    tgt = jnp.concatenate([tgt, jnp.zeros((pad,), jnp.int32)])
    edg = jnp.stack([src.reshape(NW, CPW, K), tgt.reshape(NW, CPW, K)],
                    axis=2)
    parts = _sc_scatter(Wh, edg)
    return _sum2(parts)


# speedup vs baseline: 1.7773x; 1.0843x over previous
import functools

import jax
import jax.numpy as jnp
from jax import lax
from jax.experimental import pallas as pl
from jax.experimental.pallas import tpu as pltpu
from jax.experimental.pallas import tpu_sc as plsc

N_NODES = 10000
D_OUT = 128
N_EDGES = 320000

NC = 2
NS = 16
NW = NC * NS
K = 128

CPW_A = 102
CPW_B = 56
NCHUNKS = NS * (CPW_A + CPW_B)
E_PAD = NCHUNKS * K

ACC_ROWS = 10240
SHARD = ACC_ROWS // NS
LAST_ROWS = N_NODES - (NS - 1) * SHARD


def _matmul_body(x_ref, w_ref, o_ref):
    o_ref[...] = jnp.dot(x_ref[...], w_ref[...],
                         preferred_element_type=jnp.float32)


def _matmul(X, W):
    n, d_in = X.shape
    d_out = W.shape[1]
    blk = 2000
    grid = n // blk
    return pl.pallas_call(
        _matmul_body,
        grid=(grid,),
        in_specs=[
            pl.BlockSpec((blk, d_in), lambda i: (i, 0)),
            pl.BlockSpec((d_in, d_out), lambda i: (0, 0)),
        ],
        out_specs=pl.BlockSpec((blk, d_out), lambda i: (i, 0)),
        out_shape=jax.ShapeDtypeStruct((n, d_out), jnp.float32),
    )(X, W)


def _sc_body(wh_hbm, edg_hbm, out_hbm, acc, idx_all, rows, gsem):
    cid = lax.axis_index("c")
    sid = lax.axis_index("s")
    wbase = jnp.where(cid == 0, sid * CPW_A, NS * CPW_A + sid * CPW_B)

    @pl.when(cid == 0)
    def _():
        pltpu.async_copy(edg_hbm.at[pl.ds(wbase, CPW_A)], idx_all, gsem)

    @pl.when(cid == 1)
    def _():
        pltpu.async_copy(edg_hbm.at[pl.ds(wbase, CPW_B)],
                         idx_all.at[pl.ds(0, CPW_B)], gsem)

    def _zero_row(i, carry):
        for c in range(D_OUT // 16):
            rows[i, pl.ds(c * 16, 16)] = jnp.zeros((16,), jnp.float32)
        return carry

    lax.fori_loop(0, K, _zero_row, 0)
    zbase = sid * SHARD
    for j in range(SHARD // K):
        pltpu.sync_copy(rows, acc.at[pl.ds(zbase + j * K, K), :])
    plsc.subcore_barrier()

    def _chunk(c, carry):
        pltpu.async_copy(wh_hbm.at[idx_all.at[c, 1]], rows, gsem).wait()
        pltpu.sync_copy(rows, acc.at[idx_all.at[c, 0]], add=True)
        return carry

    @pl.when(cid == 0)
    def _():
        pltpu.make_async_copy(edg_hbm.at[pl.ds(wbase, CPW_A)], idx_all,
                              gsem).wait()
        lax.fori_loop(0, CPW_A, _chunk, 0)

    @pl.when(cid == 1)
    def _():
        pltpu.make_async_copy(edg_hbm.at[pl.ds(wbase, CPW_B)],
                              idx_all.at[pl.ds(0, CPW_B)], gsem).wait()
        lax.fori_loop(0, CPW_B, _chunk, 0)

    plsc.subcore_barrier()

    rb = sid * SHARD

    @pl.when(sid < NS - 1)
    def _():
        pltpu.sync_copy(acc.at[pl.ds(rb, SHARD), :],
                        out_hbm.at[cid, pl.ds(rb, SHARD), :])

    @pl.when(sid == NS - 1)
    def _():
        pltpu.sync_copy(acc.at[pl.ds(rb, LAST_ROWS), :],
                        out_hbm.at[cid, pl.ds(rb, LAST_ROWS), :])


_sc_scatter = functools.partial(
    pl.kernel,
    out_type=jax.ShapeDtypeStruct((NC, N_NODES, D_OUT), jnp.float32),
    mesh=plsc.VectorSubcoreMesh(core_axis_name="c", subcore_axis_name="s"),
    scratch_types=[
        pltpu.VMEM_SHARED((ACC_ROWS, D_OUT), jnp.float32),
        pltpu.VMEM((CPW_A, 2, K), jnp.int32),
        pltpu.VMEM((K, D_OUT), jnp.float32),
        pltpu.SemaphoreType.DMA,
    ],
)(_sc_body)


def _sum_body(p_ref, o_ref):
    o_ref[...] = p_ref[0] + p_ref[1]


def _sum2(parts):
    _, n, d = parts.shape
    blk = 2000
    return pl.pallas_call(
        _sum_body,
        grid=(n // blk,),
        in_specs=[pl.BlockSpec((NC, blk, d), lambda i: (0, i, 0))],
        out_specs=pl.BlockSpec((blk, d), lambda i: (i, 0)),
        out_shape=jax.ShapeDtypeStruct((n, d), jnp.float32),
    )(parts)


def kernel(X, edges, W, a):
    del a
    n = X.shape[0]
    e = edges.shape[1]
    Wh = _matmul(X, W)
    src = edges[0].astype(jnp.int32)
    tgt = edges[1].astype(jnp.int32)
    pad = E_PAD - e
    src = jnp.concatenate([src, jnp.full((pad,), n, jnp.int32)])
    tgt = jnp.concatenate([tgt, jnp.zeros((pad,), jnp.int32)])
    edg = jnp.stack([src.reshape(NCHUNKS, K), tgt.reshape(NCHUNKS, K)],
                    axis=1)
    parts = _sc_scatter(Wh, edg)
    return _sum2(parts)


# speedup vs baseline: 1.8558x; 1.0442x over previous
import functools

import jax
import jax.numpy as jnp
from jax import lax
from jax.experimental import pallas as pl
from jax.experimental.pallas import tpu as pltpu
from jax.experimental.pallas import tpu_sc as plsc

N_NODES = 10000
D_OUT = 128
N_EDGES = 320000

NC = 2
NS = 16
NW = NC * NS
K = 128

CPW_A = 110
CPW_B = 48
NCHUNKS = NS * (CPW_A + CPW_B)
E_PAD = NCHUNKS * K

ACC_ROWS = 10240
SHARD = ACC_ROWS // NS
LAST_ROWS = N_NODES - (NS - 1) * SHARD


def _matmul_body(x_ref, w_ref, o_ref):
    o_ref[...] = jnp.dot(x_ref[...], w_ref[...],
                         preferred_element_type=jnp.float32)


def _matmul(X, W):
    n, d_in = X.shape
    d_out = W.shape[1]
    blk = 2000
    grid = n // blk
    return pl.pallas_call(
        _matmul_body,
        grid=(grid,),
        in_specs=[
            pl.BlockSpec((blk, d_in), lambda i: (i, 0)),
            pl.BlockSpec((d_in, d_out), lambda i: (0, 0)),
        ],
        out_specs=pl.BlockSpec((blk, d_out), lambda i: (i, 0)),
        out_shape=jax.ShapeDtypeStruct((n, d_out), jnp.float32),
    )(X, W)


def _sc_body(wh_hbm, edg_hbm, out_hbm, acc, idx_all, rows, gsem):
    cid = lax.axis_index("c")
    sid = lax.axis_index("s")
    wbase = jnp.where(cid == 0, sid * CPW_A, NS * CPW_A + sid * CPW_B)

    @pl.when(cid == 0)
    def _():
        pltpu.async_copy(edg_hbm.at[pl.ds(wbase, CPW_A)], idx_all, gsem)

    @pl.when(cid == 1)
    def _():
        pltpu.async_copy(edg_hbm.at[pl.ds(wbase, CPW_B)],
                         idx_all.at[pl.ds(0, CPW_B)], gsem)

    def _zero_row(i, carry):
        for c in range(D_OUT // 16):
            rows[i, pl.ds(c * 16, 16)] = jnp.zeros((16,), jnp.float32)
        return carry

    lax.fori_loop(0, K, _zero_row, 0)
    zbase = sid * SHARD
    for j in range(SHARD // K):
        pltpu.sync_copy(rows, acc.at[pl.ds(zbase + j * K, K), :])
    plsc.subcore_barrier()

    def _chunk(c, carry):
        pltpu.async_copy(wh_hbm.at[idx_all.at[c, 1]], rows, gsem).wait()
        pltpu.sync_copy(rows, acc.at[idx_all.at[c, 0]], add=True)
        return carry

    @pl.when(cid == 0)
    def _():
        pltpu.make_async_copy(edg_hbm.at[pl.ds(wbase, CPW_A)], idx_all,
                              gsem).wait()
        lax.fori_loop(0, CPW_A, _chunk, 0)

    @pl.when(cid == 1)
    def _():
        pltpu.make_async_copy(edg_hbm.at[pl.ds(wbase, CPW_B)],
                              idx_all.at[pl.ds(0, CPW_B)], gsem).wait()
        lax.fori_loop(0, CPW_B, _chunk, 0)

    plsc.subcore_barrier()

    rb = sid * SHARD

    @pl.when(sid < NS - 1)
    def _():
        pltpu.sync_copy(acc.at[pl.ds(rb, SHARD), :],
                        out_hbm.at[cid, pl.ds(rb, SHARD), :])

    @pl.when(sid == NS - 1)
    def _():
        pltpu.sync_copy(acc.at[pl.ds(rb, LAST_ROWS), :],
                        out_hbm.at[cid, pl.ds(rb, LAST_ROWS), :])


_sc_scatter = functools.partial(
    pl.kernel,
    out_type=jax.ShapeDtypeStruct((NC, N_NODES, D_OUT), jnp.float32),
    mesh=plsc.VectorSubcoreMesh(core_axis_name="c", subcore_axis_name="s"),
    scratch_types=[
        pltpu.VMEM_SHARED((ACC_ROWS, D_OUT), jnp.float32),
        pltpu.VMEM((CPW_A, 2, K), jnp.int32),
        pltpu.VMEM((K, D_OUT), jnp.float32),
        pltpu.SemaphoreType.DMA,
    ],
)(_sc_body)


def _sum_body(p_ref, o_ref):
    o_ref[...] = p_ref[0] + p_ref[1]


def _sum2(parts):
    _, n, d = parts.shape
    blk = 2000
    return pl.pallas_call(
        _sum_body,
        grid=(n // blk,),
        in_specs=[pl.BlockSpec((NC, blk, d), lambda i: (0, i, 0))],
        out_specs=pl.BlockSpec((blk, d), lambda i: (i, 0)),
        out_shape=jax.ShapeDtypeStruct((n, d), jnp.float32),
    )(parts)


def kernel(X, edges, W, a):
    del a
    n = X.shape[0]
    e = edges.shape[1]
    Wh = _matmul(X, W)
    src = edges[0].astype(jnp.int32)
    tgt = edges[1].astype(jnp.int32)
    pad = E_PAD - e
    src = jnp.concatenate([src, jnp.full((pad,), n, jnp.int32)])
    tgt = jnp.concatenate([tgt, jnp.zeros((pad,), jnp.int32)])
    edg = jnp.stack([src.reshape(NCHUNKS, K), tgt.reshape(NCHUNKS, K)],
                    axis=1)
    parts = _sc_scatter(Wh, edg)
    return _sum2(parts)


# speedup vs baseline: 1.8585x; 1.0015x over previous
import functools

import jax
import jax.numpy as jnp
from jax import lax
from jax.experimental import pallas as pl
from jax.experimental.pallas import tpu as pltpu
from jax.experimental.pallas import tpu_sc as plsc

N_NODES = 10000
D_OUT = 128
N_EDGES = 320000

NC = 2
NS = 16
NW = NC * NS
K = 128

CPW_A = 119
CPW_B = 39
NCHUNKS = NS * (CPW_A + CPW_B)
E_PAD = NCHUNKS * K

ACC_ROWS = 10240
SHARD = ACC_ROWS // NS
LAST_ROWS = N_NODES - (NS - 1) * SHARD


def _matmul_body(x_ref, w_ref, o_ref):
    o_ref[...] = jnp.dot(x_ref[...], w_ref[...],
                         preferred_element_type=jnp.float32)


def _matmul(X, W):
    n, d_in = X.shape
    d_out = W.shape[1]
    blk = 2000
    grid = n // blk
    return pl.pallas_call(
        _matmul_body,
        grid=(grid,),
        in_specs=[
            pl.BlockSpec((blk, d_in), lambda i: (i, 0)),
            pl.BlockSpec((d_in, d_out), lambda i: (0, 0)),
        ],
        out_specs=pl.BlockSpec((blk, d_out), lambda i: (i, 0)),
        out_shape=jax.ShapeDtypeStruct((n, d_out), jnp.float32),
    )(X, W)


def _sc_body(wh_hbm, edg_hbm, out_hbm, acc, idx_all, rows, gsem):
    cid = lax.axis_index("c")
    sid = lax.axis_index("s")
    wbase = jnp.where(cid == 0, sid * CPW_A, NS * CPW_A + sid * CPW_B)

    @pl.when(cid == 0)
    def _():
        pltpu.async_copy(edg_hbm.at[pl.ds(wbase, CPW_A)], idx_all, gsem)

    @pl.when(cid == 1)
    def _():
        pltpu.async_copy(edg_hbm.at[pl.ds(wbase, CPW_B)],
                         idx_all.at[pl.ds(0, CPW_B)], gsem)

    def _zero_row(i, carry):
        for c in range(D_OUT // 16):
            rows[i, pl.ds(c * 16, 16)] = jnp.zeros((16,), jnp.float32)
        return carry

    lax.fori_loop(0, K, _zero_row, 0)
    zbase = sid * SHARD
    for j in range(SHARD // K):
        pltpu.sync_copy(rows, acc.at[pl.ds(zbase + j * K, K), :])
    plsc.subcore_barrier()

    def _chunk(c, carry):
        pltpu.async_copy(wh_hbm.at[idx_all.at[c, 1]], rows, gsem).wait()
        pltpu.sync_copy(rows, acc.at[idx_all.at[c, 0]], add=True)
        return carry

    @pl.when(cid == 0)
    def _():
        pltpu.make_async_copy(edg_hbm.at[pl.ds(wbase, CPW_A)], idx_all,
                              gsem).wait()
        lax.fori_loop(0, CPW_A, _chunk, 0)

    @pl.when(cid == 1)
    def _():
        pltpu.make_async_copy(edg_hbm.at[pl.ds(wbase, CPW_B)],
                              idx_all.at[pl.ds(0, CPW_B)], gsem).wait()
        lax.fori_loop(0, CPW_B, _chunk, 0)

    plsc.subcore_barrier()

    rb = sid * SHARD

    @pl.when(sid < NS - 1)
    def _():
        pltpu.sync_copy(acc.at[pl.ds(rb, SHARD), :],
                        out_hbm.at[cid, pl.ds(rb, SHARD), :])

    @pl.when(sid == NS - 1)
    def _():
        pltpu.sync_copy(acc.at[pl.ds(rb, LAST_ROWS), :],
                        out_hbm.at[cid, pl.ds(rb, LAST_ROWS), :])


_sc_scatter = functools.partial(
    pl.kernel,
    out_type=jax.ShapeDtypeStruct((NC, N_NODES, D_OUT), jnp.float32),
    mesh=plsc.VectorSubcoreMesh(core_axis_name="c", subcore_axis_name="s"),
    scratch_types=[
        pltpu.VMEM_SHARED((ACC_ROWS, D_OUT), jnp.float32),
        pltpu.VMEM((CPW_A, 2, K), jnp.int32),
        pltpu.VMEM((K, D_OUT), jnp.float32),
        pltpu.SemaphoreType.DMA,
    ],
)(_sc_body)


def _sum_body(p_ref, o_ref):
    o_ref[...] = p_ref[0] + p_ref[1]


def _sum2(parts):
    _, n, d = parts.shape
    blk = 2000
    return pl.pallas_call(
        _sum_body,
        grid=(n // blk,),
        in_specs=[pl.BlockSpec((NC, blk, d), lambda i: (0, i, 0))],
        out_specs=pl.BlockSpec((blk, d), lambda i: (i, 0)),
        out_shape=jax.ShapeDtypeStruct((n, d), jnp.float32),
    )(parts)


def kernel(X, edges, W, a):
    del a
    n = X.shape[0]
    e = edges.shape[1]
    Wh = _matmul(X, W)
    src = edges[0].astype(jnp.int32)
    tgt = edges[1].astype(jnp.int32)
    pad = E_PAD - e
    src = jnp.concatenate([src, jnp.full((pad,), n, jnp.int32)])
    tgt = jnp.concatenate([tgt, jnp.zeros((pad,), jnp.int32)])
    edg = jnp.stack([src.reshape(NCHUNKS, K), tgt.reshape(NCHUNKS, K)],
                    axis=1)
    parts = _sc_scatter(Wh, edg)
    return _sum2(parts)
